# baseline (speedup 1.0000x reference)
import jax
import jax.numpy as jnp
from jax import lax
from jax.experimental import pallas as pl
from jax.experimental.pallas import tpu as pltpu


def _body(tt_ref, x_ref, w2_ref, o_ref):
    tm = tt_ref[0]
    w = w2_ref[...]
    lane = jax.lax.broadcasted_iota(jnp.int32, w.shape, 1)
    for q in range(16):
        colq = jnp.sum(
            jnp.where(lane == 8 * q + tm, w, 0.0), axis=1, keepdims=True
        )
        o_ref[:, q, :] = x_ref[:, q, :] - colq


def kernel(x, weight, token_type):
    n, d = x.shape
    bn = 2048
    bnk = bn // 16
    t = jnp.asarray(token_type, jnp.int32)
    t0 = (t // 8) * 8
    w8 = lax.dynamic_slice(weight, (jnp.int32(0), t0), (n, 8))
    w2 = w8.reshape(n // 16, 128)
    x3 = x.reshape(n // 16, 16, d)
    tm = (t % 8).reshape(1)
    out3 = pl.pallas_call(
        _body,
        grid=(n // bn,),
        in_specs=[
            pl.BlockSpec(memory_space=pltpu.SMEM),
            pl.BlockSpec((bnk, 16, d), lambda i: (i, 0, 0)),
            pl.BlockSpec((bnk, 128), lambda i: (i, 0)),
        ],
        out_specs=pl.BlockSpec((bnk, 16, d), lambda i: (i, 0, 0)),
        out_shape=jax.ShapeDtypeStruct((n // 16, 16, d), jnp.float32),
    )(tm, x3, w2)
    return out3.reshape(n, d)
